# baseline (device time: 23860 ns/iter reference)
import jax
import jax.numpy as jnp
from jax import lax
from jax.experimental import pallas as pl
from jax.experimental.pallas import tpu as pltpu

X, Y, Z = 2, 4, 4
NXY = X * Y
B, H, D, BS = 8, 8, 128, 16
HB, HD = H * B, H * D
NB = 512
LOCAL_PAGES = 512
PPC = LOCAL_PAGES // NXY
TOK = PPC * BS
SCALE = D ** -0.5
N_PEERS = (NXY - 1) + (Z - 1)


def kernel(Q, K, V, bt, lens):
    Qs = Q[:, 0, :, :]
    Qt = jnp.transpose(Qs, (1, 2, 0))
    eye = jnp.eye(H, dtype=jnp.float32)
    Qbd = (eye[:, None, :, None] * Qt[:, :, None, :]).reshape(HD, HB)
    lens2 = lens.reshape(B, 1)

    def body(qbd_ref, k_ref, v_ref, bt_ref, lens_ref, out_ref,
             kbuf, vbuf, ga_num, ga_den, gb_num, gb_den,
             copy_sems, a_send, a_recv, b_send, b_recv):
        my_x = lax.axis_index("x")
        my_y = lax.axis_index("y")
        my_z = lax.axis_index("z")
        xy = my_x * Y + my_y
        off = xy * PPC
        base = my_z * LOCAL_PAGES + off

        barrier = pltpu.get_barrier_semaphore()
        for s in range(NXY):
            px, py = s // Y, s % Y

            @pl.when(xy != s)
            def _sig(px=px, py=py):
                pl.semaphore_signal(
                    barrier, inc=1, device_id=(px, py, my_z),
                    device_id_type=pl.DeviceIdType.MESH)
        for zoff in range(1, Z):
            peer = (my_z + zoff) % Z
            pl.semaphore_signal(
                barrier, inc=1, device_id=(my_x, my_y, peer),
                device_id_type=pl.DeviceIdType.MESH)

        kcopies, vcopies = [], []
        for h in range(H):
            ckh = pltpu.make_async_copy(
                k_ref.at[pl.ds(off, PPC), :, h], kbuf.at[h], copy_sems.at[0, h])
            cvh = pltpu.make_async_copy(
                v_ref.at[pl.ds(off, PPC), :, h], vbuf.at[h], copy_sems.at[1, h])
            ckh.start()
            cvh.start()
            kcopies.append(ckh)
            vcopies.append(cvh)

        bt_ = bt_ref[...]
        pid3 = base + lax.broadcasted_iota(jnp.int32, (PPC, B, NB), 0)
        slot3 = lax.broadcasted_iota(jnp.int32, (PPC, B, NB), 2)
        hit = (bt_[None, :, :] == pid3) & (slot3 < lens_ref[...][None, :, :])
        counts_t = jnp.sum(hit.astype(jnp.float32), axis=2)
        expand = (lax.broadcasted_iota(jnp.int32, (TOK, PPC), 0) // BS
                  == lax.broadcasted_iota(jnp.int32, (TOK, PPC), 1)
                  ).astype(jnp.float32)
        tile = (lax.broadcasted_iota(jnp.int32, (B, HB), 1) % B
                == lax.broadcasted_iota(jnp.int32, (B, HB), 0)
                ).astype(jnp.float32)
        w2 = jnp.dot(expand, jnp.dot(counts_t, tile))

        s2 = jnp.zeros((TOK, HB), jnp.float32)
        for h in range(H):
            kcopies[h].wait()
            k_h = kbuf[h].reshape(TOK, D)
            s2 = s2 + lax.dot_general(
                k_h, qbd_ref[h * D:(h + 1) * D, :],
                (((1,), (0,)), ((), ())))
        p2 = jnp.exp(s2 * SCALE) * w2
        for h in range(H):
            vcopies[h].wait()
            v_h = vbuf[h].reshape(TOK, D)
            ga_num[xy, h] = lax.dot_general(
                p2[:, h * B:(h + 1) * B], v_h,
                (((0,), (0,)), ((), ())))
        den_col = lax.dot_general(
            p2, jnp.ones((TOK, 1), jnp.float32),
            (((0,), (0,)), ((), ())))
        ga_den[xy] = den_col.reshape(H, B, 1)

        pl.semaphore_wait(barrier, N_PEERS)

        a_sends = []
        for s in range(NXY):
            px, py = s // Y, s % Y
            for kind, buf in ((0, ga_num), (1, ga_den)):
                rd = pltpu.make_async_remote_copy(
                    src_ref=buf.at[xy],
                    dst_ref=buf.at[xy],
                    send_sem=a_send.at[kind, s],
                    recv_sem=a_recv.at[kind, xy],
                    device_id=(px, py, my_z),
                    device_id_type=pl.DeviceIdType.MESH,
                )
                a_sends.append((s, rd))

                @pl.when(xy != s)
                def _start(rd=rd):
                    rd.start()

        for s in range(NXY):
            px, py = s // Y, s % Y
            for kind, buf in ((0, ga_num), (1, ga_den)):
                rr = pltpu.make_async_remote_copy(
                    src_ref=buf.at[s],
                    dst_ref=buf.at[s],
                    send_sem=a_send.at[kind, s],
                    recv_sem=a_recv.at[kind, s],
                    device_id=(px, py, my_z),
                    device_id_type=pl.DeviceIdType.MESH,
                )

                @pl.when(xy != s)
                def _waitr(rr=rr):
                    rr.wait_recv()

        for s, rd in a_sends:
            @pl.when(xy != s)
            def _waits(rd=rd):
                rd.wait_send()

        gb_num[my_z] = jnp.sum(ga_num[...], axis=0)
        gb_den[my_z] = jnp.sum(ga_den[...], axis=0)

        b_sends = []
        for zoff in range(1, Z):
            peer = (my_z + zoff) % Z
            for kind, buf in ((0, gb_num), (1, gb_den)):
                rd = pltpu.make_async_remote_copy(
                    src_ref=buf.at[my_z],
                    dst_ref=buf.at[my_z],
                    send_sem=b_send.at[kind, peer],
                    recv_sem=b_recv.at[kind, my_z],
                    device_id=(my_x, my_y, peer),
                    device_id_type=pl.DeviceIdType.MESH,
                )
                rd.start()
                b_sends.append(rd)
        for zoff in range(1, Z):
            src = (my_z + zoff) % Z
            for kind, buf in ((0, gb_num), (1, gb_den)):
                rr = pltpu.make_async_remote_copy(
                    src_ref=buf.at[src],
                    dst_ref=buf.at[src],
                    send_sem=b_send.at[kind, src],
                    recv_sem=b_recv.at[kind, src],
                    device_id=(my_x, my_y, src),
                    device_id_type=pl.DeviceIdType.MESH,
                )
                rr.wait_recv()
        for rd in b_sends:
            rd.wait_send()

        num = jnp.sum(gb_num[...], axis=0)
        den = jnp.sum(gb_den[...], axis=0)
        o = num / den
        out_ref[...] = jnp.swapaxes(o, 0, 1)[:, None, :, :]

    return pl.pallas_call(
        body,
        in_specs=[
            pl.BlockSpec(memory_space=pltpu.MemorySpace.VMEM),
            pl.BlockSpec(memory_space=pltpu.MemorySpace.HBM),
            pl.BlockSpec(memory_space=pltpu.MemorySpace.HBM),
            pl.BlockSpec(memory_space=pltpu.MemorySpace.VMEM),
            pl.BlockSpec(memory_space=pltpu.MemorySpace.VMEM),
        ],
        out_specs=pl.BlockSpec(memory_space=pltpu.MemorySpace.VMEM),
        out_shape=jax.ShapeDtypeStruct((B, 1, H, D), jnp.float32),
        scratch_shapes=[
            pltpu.VMEM((H, PPC, BS, D), jnp.float32),
            pltpu.VMEM((H, PPC, BS, D), jnp.float32),
            pltpu.VMEM((NXY, H, B, D), jnp.float32),
            pltpu.VMEM((NXY, H, B, 1), jnp.float32),
            pltpu.VMEM((Z, H, B, D), jnp.float32),
            pltpu.VMEM((Z, H, B, 1), jnp.float32),
            pltpu.SemaphoreType.DMA((2, H)),
            pltpu.SemaphoreType.DMA((2, NXY)),
            pltpu.SemaphoreType.DMA((2, NXY)),
            pltpu.SemaphoreType.DMA((2, Z)),
            pltpu.SemaphoreType.DMA((2, Z)),
        ],
        compiler_params=pltpu.CompilerParams(collective_id=0),
    )(Qbd, K, V, bt, lens2)
